# baseline (speedup 1.0000x reference)
import functools

import jax
import jax.numpy as jnp
from jax import lax
from jax.experimental import pallas as pl
from jax.experimental.pallas import tpu as pltpu
from jax.experimental.pallas import tpu_sc as plsc

_NC = 2
_NS = 16
_NW = _NC * _NS

_GW = 400
_SW = 200


def _gather_sc(x, src, dst):
    N, C = x.shape
    E = src.shape[0]
    epw = E // _NW
    steps = epw // _GW
    mesh = plsc.VectorSubcoreMesh(core_axis_name="c", subcore_axis_name="s")

    @functools.partial(
        pl.kernel,
        out_type=(
            jax.ShapeDtypeStruct((E, C), jnp.float32),
            jax.ShapeDtypeStruct((E, C), jnp.float32),
        ),
        mesh=mesh,
        scratch_types=[
            pltpu.VMEM((_GW,), jnp.int32),
            pltpu.VMEM((_GW,), jnp.int32),
            pltpu.VMEM((_GW, C), jnp.float32),
            pltpu.VMEM((_GW, C), jnp.float32),
            pltpu.SemaphoreType.DMA,
            pltpu.SemaphoreType.DMA,
        ],
    )
    def gather_kernel(x_hbm, src_hbm, dst_hbm, xs_hbm, xd_hbm,
                      idx_s, idx_d, rows_s, rows_d, sem_s, sem_d):
        cid = lax.axis_index("c")
        sid = lax.axis_index("s")
        wid = sid * _NC + cid
        base = wid * epw

        def step(w, carry):
            off = base + w * _GW
            pltpu.sync_copy(src_hbm.at[pl.ds(off, _GW)], idx_s)
            pltpu.sync_copy(dst_hbm.at[pl.ds(off, _GW)], idx_d)
            cs = pltpu.async_copy(x_hbm.at[idx_s], rows_s, sem_s)
            cd = pltpu.async_copy(x_hbm.at[idx_d], rows_d, sem_d)
            cs.wait()
            cd.wait()
            pltpu.sync_copy(rows_s, xs_hbm.at[pl.ds(off, _GW)])
            pltpu.sync_copy(rows_d, xd_hbm.at[pl.ds(off, _GW)])
            return carry

        lax.fori_loop(0, steps, step, 0)

    return gather_kernel(x, src, dst)


def _scatter_sc(messages, dst, init, num_nodes):
    nmsg = len(messages)
    ec, C = messages[0].shape
    epw = ec // _NW
    steps = epw // _SW
    npad = ((num_nodes + 8 * _NS - 1) // (8 * _NS)) * (8 * _NS)
    npt = npad // _NS
    mesh = plsc.VectorSubcoreMesh(core_axis_name="c", subcore_axis_name="s")

    @functools.partial(
        pl.kernel,
        out_type=jax.ShapeDtypeStruct((_NC, npad, C), jnp.float32),
        mesh=mesh,
        scratch_types=[
            pltpu.VMEM((_SW,), jnp.int32),
            pltpu.VMEM((_SW, C), jnp.float32),
            pltpu.VMEM_SHARED((npad, C), jnp.float32),
        ],
    )
    def scatter_kernel(dst_hbm, init_hbm, *refs):
        msg_hbms = refs[:nmsg]
        out_hbm = refs[nmsg]
        idx_v, rows_v, acc = refs[nmsg + 1:]
        cid = lax.axis_index("c")
        sid = lax.axis_index("s")
        wid = sid * _NC + cid
        pltpu.sync_copy(init_hbm.at[cid, pl.ds(sid * npt, npt)],
                        acc.at[pl.ds(sid * npt, npt)])
        plsc.subcore_barrier()
        base = wid * epw

        for k in range(nmsg):
            def step(w, carry, k=k):
                off = base + w * _SW
                pltpu.sync_copy(dst_hbm.at[pl.ds(k * ec + off, _SW)], idx_v)
                pltpu.sync_copy(msg_hbms[k].at[pl.ds(off, _SW)], rows_v)
                pltpu.sync_copy(rows_v, acc.at[idx_v], add=True)
                return carry

            lax.fori_loop(0, steps, step, 0)
        plsc.subcore_barrier()
        pltpu.sync_copy(acc.at[pl.ds(sid * npt, npt)],
                        out_hbm.at[cid, pl.ds(sid * npt, npt)])

    return scatter_kernel(dst, init, *messages)


def _message_tc(xs, xd, ef, W1a, W1b, W1e, bm1, Wm2, bm2):
    E, C = xs.shape
    ED = ef.shape[1]
    MH = W1a.shape[1]
    blk = 8000
    grid = (E // blk,)

    def body(xs_ref, xd_ref, ef_ref, w1a_ref, w1b_ref, w1e_ref, bm1_ref,
             wm2_ref, bm2_ref, out_ref):
        pre = (
            jnp.dot(xs_ref[...].astype(jnp.bfloat16), w1a_ref[...],
                    preferred_element_type=jnp.float32)
            + jnp.dot(xd_ref[...].astype(jnp.bfloat16), w1b_ref[...],
                      preferred_element_type=jnp.float32)
            + jnp.dot(jnp.log1p(ef_ref[...]), w1e_ref[...],
                      preferred_element_type=jnp.float32)
            + bm1_ref[...]
        )
        h = jnp.maximum(pre, 0.0).astype(jnp.bfloat16)
        out_ref[...] = (
            jnp.dot(h, wm2_ref[...], preferred_element_type=jnp.float32)
            + bm2_ref[...]
        )

    return pl.pallas_call(
        body,
        grid=grid,
        in_specs=[
            pl.BlockSpec((blk, C), lambda i: (i, 0)),
            pl.BlockSpec((blk, C), lambda i: (i, 0)),
            pl.BlockSpec((blk, ED), lambda i: (i, 0)),
            pl.BlockSpec((C, MH), lambda i: (0, 0)),
            pl.BlockSpec((C, MH), lambda i: (0, 0)),
            pl.BlockSpec((ED, MH), lambda i: (0, 0)),
            pl.BlockSpec((1, MH), lambda i: (0, 0)),
            pl.BlockSpec((MH, C), lambda i: (0, 0)),
            pl.BlockSpec((1, C), lambda i: (0, 0)),
        ],
        out_specs=pl.BlockSpec((blk, C), lambda i: (i, 0)),
        out_shape=jax.ShapeDtypeStruct((E, C), jnp.float32),
    )(xs, xd, ef, W1a, W1b, W1e, bm1.reshape(1, MH), Wm2, bm2.reshape(1, C))


def _update_tc(x, parts, Wu1a, Wu1b, bu1, Wu2, bu2):
    N, C = x.shape
    UH = Wu1a.shape[1]
    blk = 2000
    grid = (N // blk,)

    def body(x_ref, p0_ref, p1_ref, wu1a_ref, wu1b_ref, bu1_ref, wu2_ref,
             bu2_ref, out_ref):
        agg = p0_ref[...] + p1_ref[...]
        pre = (
            jnp.dot(x_ref[...], wu1a_ref[...], preferred_element_type=jnp.float32)
            + jnp.dot(agg, wu1b_ref[...], preferred_element_type=jnp.float32)
            + bu1_ref[...]
        )
        h = jnp.maximum(pre, 0.0)
        out_ref[...] = (
            jnp.dot(h, wu2_ref[...], preferred_element_type=jnp.float32)
            + bu2_ref[...]
        )

    return pl.pallas_call(
        body,
        grid=grid,
        in_specs=[
            pl.BlockSpec((blk, C), lambda i: (i, 0)),
            pl.BlockSpec((blk, C), lambda i: (i, 0)),
            pl.BlockSpec((blk, C), lambda i: (i, 0)),
            pl.BlockSpec((C, UH), lambda i: (0, 0)),
            pl.BlockSpec((C, UH), lambda i: (0, 0)),
            pl.BlockSpec((1, UH), lambda i: (0, 0)),
            pl.BlockSpec((UH, C), lambda i: (0, 0)),
            pl.BlockSpec((1, C), lambda i: (0, 0)),
        ],
        out_specs=pl.BlockSpec((blk, C), lambda i: (i, 0)),
        out_shape=jax.ShapeDtypeStruct((N, C), jnp.float32),
    )(x, parts[0], parts[1], Wu1a, Wu1b, bu1.reshape(1, UH), Wu2,
      bu2.reshape(1, C))


def kernel(x, edge_index, edge_features, Wm1, bm1, Wm2, bm2, Wu1, bu1, Wu2, bu2):
    N, C = x.shape
    E = edge_index.shape[1]
    src = edge_index[0].astype(jnp.int32)
    dst = edge_index[1].astype(jnp.int32)

    nchunk = 5
    ce = E // nchunk
    npad = ((N + 8 * _NS - 1) // (8 * _NS)) * (8 * _NS)
    parts = jnp.zeros((_NC, npad, C), jnp.float32)
    groups = [(0, 1, 2), (3, 4)]
    msgs = []
    for k in range(nchunk):
        sl = slice(k * ce, (k + 1) * ce)
        xs, xd = _gather_sc(x, src[sl], dst[sl])
        msgs.append(_message_tc(xs, xd, edge_features[sl],
                                Wm1[:C].astype(jnp.bfloat16),
                                Wm1[C:2 * C].astype(jnp.bfloat16),
                                Wm1[2 * C:], bm1,
                                Wm2.astype(jnp.bfloat16), bm2))
    for g in groups:
        dsl = dst[g[0] * ce:(g[-1] + 1) * ce]
        parts = _scatter_sc([msgs[k] for k in g], dsl, parts, N)
    return _update_tc(x, parts[:, :N], Wu1[:C], Wu1[C:], bu1, Wu2, bu2)


# speedup vs baseline: 1.0023x; 1.0023x over previous
import functools

import jax
import jax.numpy as jnp
from jax import lax
from jax.experimental import pallas as pl
from jax.experimental.pallas import tpu as pltpu
from jax.experimental.pallas import tpu_sc as plsc

_NC = 2
_NS = 16
_NW = _NC * _NS

_GW = 400
_SW = 200


def _gather_sc(x, src, dst):
    N, C = x.shape
    E = src.shape[0]
    epw = E // _NW
    steps = epw // _GW
    mesh = plsc.VectorSubcoreMesh(core_axis_name="c", subcore_axis_name="s")

    @functools.partial(
        pl.kernel,
        out_type=(
            jax.ShapeDtypeStruct((E, C), jnp.float32),
            jax.ShapeDtypeStruct((E, C), jnp.float32),
        ),
        mesh=mesh,
        scratch_types=[
            pltpu.VMEM((_GW,), jnp.int32),
            pltpu.VMEM((_GW,), jnp.int32),
            pltpu.VMEM((_GW, C), jnp.float32),
            pltpu.VMEM((_GW, C), jnp.float32),
            pltpu.SemaphoreType.DMA,
            pltpu.SemaphoreType.DMA,
        ],
    )
    def gather_kernel(x_hbm, src_hbm, dst_hbm, xs_hbm, xd_hbm,
                      idx_s, idx_d, rows_s, rows_d, sem_s, sem_d):
        cid = lax.axis_index("c")
        sid = lax.axis_index("s")
        wid = sid * _NC + cid
        base = wid * epw

        def step(w, carry):
            off = base + w * _GW
            pltpu.sync_copy(src_hbm.at[pl.ds(off, _GW)], idx_s)
            pltpu.sync_copy(dst_hbm.at[pl.ds(off, _GW)], idx_d)
            cs = pltpu.async_copy(x_hbm.at[idx_s], rows_s, sem_s)
            cd = pltpu.async_copy(x_hbm.at[idx_d], rows_d, sem_d)
            cs.wait()
            cd.wait()
            pltpu.sync_copy(rows_s, xs_hbm.at[pl.ds(off, _GW)])
            pltpu.sync_copy(rows_d, xd_hbm.at[pl.ds(off, _GW)])
            return carry

        lax.fori_loop(0, steps, step, 0)

    return gather_kernel(x, src, dst)


def _scatter_sc(messages, dst, init, num_nodes):
    nmsg = len(messages)
    ec, C = messages[0].shape
    epw = ec // _NW
    steps = epw // _SW
    npad = ((num_nodes + 8 * _NS - 1) // (8 * _NS)) * (8 * _NS)
    npt = npad // _NS
    mesh = plsc.VectorSubcoreMesh(core_axis_name="c", subcore_axis_name="s")

    @functools.partial(
        pl.kernel,
        out_type=jax.ShapeDtypeStruct((_NC, npad, C), jnp.float32),
        mesh=mesh,
        scratch_types=[
            pltpu.VMEM((_SW,), jnp.int32),
            pltpu.VMEM((_SW, C), jnp.float32),
            pltpu.VMEM_SHARED((npad, C), jnp.float32),
        ],
    )
    def scatter_kernel(dst_hbm, init_hbm, *refs):
        msg_hbms = refs[:nmsg]
        out_hbm = refs[nmsg]
        idx_v, rows_v, acc = refs[nmsg + 1:]
        cid = lax.axis_index("c")
        sid = lax.axis_index("s")
        wid = sid * _NC + cid
        pltpu.sync_copy(init_hbm.at[cid, pl.ds(sid * npt, npt)],
                        acc.at[pl.ds(sid * npt, npt)])
        plsc.subcore_barrier()
        base = wid * epw

        for k in range(nmsg):
            def step(w, carry, k=k):
                off = base + w * _SW
                pltpu.sync_copy(dst_hbm.at[pl.ds(k * ec + off, _SW)], idx_v)
                pltpu.sync_copy(msg_hbms[k].at[pl.ds(off, _SW)], rows_v)
                pltpu.sync_copy(rows_v, acc.at[idx_v], add=True)
                return carry

            lax.fori_loop(0, steps, step, 0)
        plsc.subcore_barrier()
        pltpu.sync_copy(acc.at[pl.ds(sid * npt, npt)],
                        out_hbm.at[cid, pl.ds(sid * npt, npt)])

    return scatter_kernel(dst, init, *messages)


def _message_tc(xs, xd, ef, W1a, W1b, W1e, bm1, Wm2, bm2):
    E, C = xs.shape
    ED = ef.shape[1]
    MH = W1a.shape[1]
    blk = 4000
    grid = (E // blk,)

    def body(xs_ref, xd_ref, ef_ref, w1a_ref, w1b_ref, w1e_ref, bm1_ref,
             wm2_ref, bm2_ref, out_ref):
        pre = (
            jnp.dot(xs_ref[...].astype(jnp.bfloat16), w1a_ref[...],
                    preferred_element_type=jnp.float32)
            + jnp.dot(xd_ref[...].astype(jnp.bfloat16), w1b_ref[...],
                      preferred_element_type=jnp.float32)
            + jnp.dot(jnp.log1p(ef_ref[...]), w1e_ref[...],
                      preferred_element_type=jnp.float32)
            + bm1_ref[...]
        )
        h = jnp.maximum(pre, 0.0).astype(jnp.bfloat16)
        out_ref[...] = (
            jnp.dot(h, wm2_ref[...], preferred_element_type=jnp.float32)
            + bm2_ref[...]
        )

    return pl.pallas_call(
        body,
        grid=grid,
        in_specs=[
            pl.BlockSpec((blk, C), lambda i: (i, 0)),
            pl.BlockSpec((blk, C), lambda i: (i, 0)),
            pl.BlockSpec((blk, ED), lambda i: (i, 0)),
            pl.BlockSpec((C, MH), lambda i: (0, 0)),
            pl.BlockSpec((C, MH), lambda i: (0, 0)),
            pl.BlockSpec((ED, MH), lambda i: (0, 0)),
            pl.BlockSpec((1, MH), lambda i: (0, 0)),
            pl.BlockSpec((MH, C), lambda i: (0, 0)),
            pl.BlockSpec((1, C), lambda i: (0, 0)),
        ],
        out_specs=pl.BlockSpec((blk, C), lambda i: (i, 0)),
        out_shape=jax.ShapeDtypeStruct((E, C), jnp.float32),
    )(xs, xd, ef, W1a, W1b, W1e, bm1.reshape(1, MH), Wm2, bm2.reshape(1, C))


def _update_tc(x, parts, Wu1a, Wu1b, bu1, Wu2, bu2):
    N, C = x.shape
    UH = Wu1a.shape[1]
    blk = 2000
    grid = (N // blk,)

    def body(x_ref, p0_ref, p1_ref, wu1a_ref, wu1b_ref, bu1_ref, wu2_ref,
             bu2_ref, out_ref):
        agg = p0_ref[...] + p1_ref[...]
        pre = (
            jnp.dot(x_ref[...], wu1a_ref[...], preferred_element_type=jnp.float32)
            + jnp.dot(agg, wu1b_ref[...], preferred_element_type=jnp.float32)
            + bu1_ref[...]
        )
        h = jnp.maximum(pre, 0.0)
        out_ref[...] = (
            jnp.dot(h, wu2_ref[...], preferred_element_type=jnp.float32)
            + bu2_ref[...]
        )

    return pl.pallas_call(
        body,
        grid=grid,
        in_specs=[
            pl.BlockSpec((blk, C), lambda i: (i, 0)),
            pl.BlockSpec((blk, C), lambda i: (i, 0)),
            pl.BlockSpec((blk, C), lambda i: (i, 0)),
            pl.BlockSpec((C, UH), lambda i: (0, 0)),
            pl.BlockSpec((C, UH), lambda i: (0, 0)),
            pl.BlockSpec((1, UH), lambda i: (0, 0)),
            pl.BlockSpec((UH, C), lambda i: (0, 0)),
            pl.BlockSpec((1, C), lambda i: (0, 0)),
        ],
        out_specs=pl.BlockSpec((blk, C), lambda i: (i, 0)),
        out_shape=jax.ShapeDtypeStruct((N, C), jnp.float32),
    )(x, parts[0], parts[1], Wu1a, Wu1b, bu1.reshape(1, UH), Wu2,
      bu2.reshape(1, C))


def kernel(x, edge_index, edge_features, Wm1, bm1, Wm2, bm2, Wu1, bu1, Wu2, bu2):
    N, C = x.shape
    E = edge_index.shape[1]
    src = edge_index[0].astype(jnp.int32)
    dst = edge_index[1].astype(jnp.int32)

    nchunk = 5
    ce = E // nchunk
    npad = ((N + 8 * _NS - 1) // (8 * _NS)) * (8 * _NS)
    parts = jnp.zeros((_NC, npad, C), jnp.float32)
    groups = [(0, 1, 2), (3, 4)]
    msgs = []
    for k in range(nchunk):
        sl = slice(k * ce, (k + 1) * ce)
        xs, xd = _gather_sc(x, src[sl], dst[sl])
        msgs.append(_message_tc(xs, xd, edge_features[sl],
                                Wm1[:C].astype(jnp.bfloat16),
                                Wm1[C:2 * C].astype(jnp.bfloat16),
                                Wm1[2 * C:], bm1,
                                Wm2.astype(jnp.bfloat16), bm2))
    for g in groups:
        dsl = dst[g[0] * ce:(g[-1] + 1) * ce]
        parts = _scatter_sc([msgs[k] for k in g], dsl, parts, N)
    return _update_tc(x, parts[:, :N], Wu1[:C], Wu1[C:], bu1, Wu2, bu2)
